# baseline (speedup 1.0000x reference)
import functools

import jax
import jax.numpy as jnp
from jax import lax
from jax.experimental import pallas as pl
from jax.experimental.pallas import tpu as pltpu
from jax.experimental.pallas import tpu_sc as plsc

WINDOW = 5


@functools.cache
def _make_sc_stage(B, L, V, D):
    info = plsc.get_sparse_core_info()
    NC, NS = info.num_cores, info.num_subcores
    NW = NC * NS
    assert B % NW == 0
    seq_per_w = B // NW
    rows_per_w = seq_per_w * L
    CH = 80
    assert rows_per_w % CH == 0
    n_ch = rows_per_w // CH

    mesh = plsc.VectorSubcoreMesh(core_axis_name="c", subcore_axis_name="s")

    @functools.partial(
        pl.kernel,
        mesh=mesh,
        compiler_params=pltpu.CompilerParams(use_tc_tiling_on_sc=False),
        out_type=jax.ShapeDtypeStruct((B * L, D), jnp.float32),
        scratch_types=[
            pltpu.VMEM((n_ch, CH), jnp.int32),
            pltpu.VMEM((rows_per_w, D), jnp.float32),
            pltpu.VMEM((L + 1, D), jnp.float32),
            pltpu.VMEM((rows_per_w, D), jnp.float32),
            pltpu.SemaphoreType.DMA,
        ],
    )
    def sc_kernel(ids_hbm, table_hbm, out_hbm, idx_v, rows_v, cum_v, grp_v,
                  sem):
        wid = lax.axis_index("s") * NC + lax.axis_index("c")
        base = wid * rows_per_w

        pltpu.sync_copy(ids_hbm.at[wid], idx_v)
        copies = []
        for j in range(n_ch):
            copies.append(
                pltpu.async_copy(
                    table_hbm.at[idx_v.at[j]],
                    rows_v.at[pl.ds(j * CH, CH)],
                    sem,
                )
            )
        for cp in copies:
            cp.wait()

        zero = jnp.zeros((D,), jnp.float32)

        def seq_body(s, _):
            row0 = s * L
            cum_v[0, :] = zero

            def cum_body(t, acc):
                acc = acc + rows_v[row0 + t, :]
                cum_v[t + 1, :] = acc
                return acc

            lax.fori_loop(0, L, cum_body, zero)

            def out_body(t, carry):
                hi = jnp.where(t + WINDOW > L, L - 1, t + WINDOW)
                lo = jnp.where(t < WINDOW, 1, t - WINDOW)
                interior = jnp.logical_and(t >= WINDOW, t + WINDOW <= L)
                cmask = jnp.where(interior, 1.0, 0.0).astype(jnp.float32)
                g = cum_v[hi, :] - cum_v[lo, :] - cmask * rows_v[row0 + t, :]
                grp_v[row0 + t, :] = g
                return carry

            lax.fori_loop(0, L, out_body, 0)
            return _

        lax.fori_loop(0, seq_per_w, seq_body, 0)
        pltpu.sync_copy(grp_v, out_hbm.at[pl.ds(base, rows_per_w)])

    def run(ids, table):
        ids3 = ids.reshape(NW, n_ch, CH)
        return sc_kernel(ids3, table)

    return run


@functools.cache
def _make_tc_stage(M, V, D, MB=512):
    assert M % MB == 0

    def mm_body(x_ref, w_ref, b_ref, o_ref):
        o_ref[...] = (
            lax.dot_general(
                x_ref[...], w_ref[...],
                (((1,), (0,)), ((), ())),
                preferred_element_type=jnp.float32,
            )
            + b_ref[...]
        )

    call = pl.pallas_call(
        mm_body,
        grid=(M // MB,),
        in_specs=[
            pl.BlockSpec((MB, D), lambda i: (i, 0)),
            pl.BlockSpec((D, V), lambda i: (0, 0)),
            pl.BlockSpec((1, V), lambda i: (0, 0)),
        ],
        out_specs=pl.BlockSpec((MB, V), lambda i: (i, 0)),
        out_shape=jax.ShapeDtypeStruct((M, V), jnp.float32),
    )

    def run(x, W, b):
        return call(x, W.T, b.reshape(1, V))

    return run


def kernel(ids, emb_table, W, b):
    B, L = ids.shape
    V, D = emb_table.shape
    ids = ids.astype(jnp.int32)
    grouped = _make_sc_stage(B, L, V, D)(ids, emb_table)
    out = _make_tc_stage(B * L, V, D)(grouped, W, b)
    return out.reshape(B, L, V)


# speedup vs baseline: 1.0014x; 1.0014x over previous
import functools

import jax
import jax.numpy as jnp
from jax import lax
from jax.experimental import pallas as pl
from jax.experimental.pallas import tpu as pltpu
from jax.experimental.pallas import tpu_sc as plsc

WINDOW = 5
LANES = 128


@functools.cache
def _make_sc_stage(B, L, V, D):
    info = plsc.get_sparse_core_info()
    NC, NS = info.num_cores, info.num_subcores
    NW = NC * NS
    assert B % NW == 0
    seq_per_w = B // NW
    rows_per_w = seq_per_w * L
    n_ch = -(-rows_per_w // LANES)
    rows_pad = n_ch * LANES
    SEQ_CHUNK = 8
    assert seq_per_w % SEQ_CHUNK == 0
    n_out_ch = seq_per_w // SEQ_CHUNK
    grp_rows = SEQ_CHUNK * L

    mesh = plsc.VectorSubcoreMesh(core_axis_name="c", subcore_axis_name="s")

    @functools.partial(
        pl.kernel,
        mesh=mesh,
        compiler_params=pltpu.CompilerParams(use_tc_tiling_on_sc=False),
        out_type=jax.ShapeDtypeStruct((B * L, LANES), jnp.float32),
        scratch_types=[
            pltpu.VMEM((n_ch, LANES), jnp.int32),
            pltpu.VMEM((rows_pad, D), jnp.float32),
            pltpu.VMEM((L + 1, D), jnp.float32),
            pltpu.VMEM((grp_rows, LANES), jnp.float32),
            pltpu.SemaphoreType.DMA,
        ],
    )
    def sc_kernel(ids_hbm, table_hbm, out_hbm, idx_v, rows_v, cum_v, grp_v,
                  sem):
        wid = lax.axis_index("s") * NC + lax.axis_index("c")
        base = wid * rows_per_w

        pltpu.sync_copy(ids_hbm.at[wid], idx_v)
        copies = []
        for j in range(n_ch):
            copies.append(
                pltpu.async_copy(
                    table_hbm.at[idx_v.at[j]],
                    rows_v.at[pl.ds(j * LANES, LANES)],
                    sem,
                )
            )

        zero = jnp.zeros((D,), jnp.float32)

        def zero_body(t, carry):
            for k in range(LANES // D):
                grp_v[t, pl.ds(k * D, D)] = zero
            return carry

        lax.fori_loop(0, grp_rows, zero_body, 0)

        for cp in copies:
            cp.wait()

        def seq_body(s8, c):
            row0 = (c * SEQ_CHUNK + s8) * L
            lrow0 = s8 * L
            cum_v[0, :] = zero

            def cum_body(t, acc):
                acc = acc + rows_v[row0 + t, :]
                cum_v[t + 1, :] = acc
                return acc

            lax.fori_loop(0, L, cum_body, zero)

            def out_body(t, carry):
                hi = jnp.where(t + WINDOW > L, L - 1, t + WINDOW)
                lo = jnp.where(t < WINDOW, 1, t - WINDOW)
                interior = jnp.logical_and(t >= WINDOW, t + WINDOW <= L)
                cmask = jnp.where(interior, 1.0, 0.0).astype(jnp.float32)
                g = cum_v[hi, :] - cum_v[lo, :] - cmask * rows_v[row0 + t, :]
                grp_v[lrow0 + t, pl.ds(0, D)] = g
                return carry

            lax.fori_loop(0, L, out_body, 0)
            return c

        for c in range(n_out_ch):
            lax.fori_loop(0, SEQ_CHUNK, seq_body, c)
            pltpu.sync_copy(grp_v, out_hbm.at[pl.ds(base + c * grp_rows,
                                                    grp_rows)])

    def run(ids, table):
        flat = ids.reshape(NW, rows_per_w)
        ids3 = jnp.pad(flat, ((0, 0), (0, rows_pad - rows_per_w))).reshape(
            NW, n_ch, LANES)
        return sc_kernel(ids3, table)

    return run


@functools.cache
def _make_tc_stage(M, V, D, MB=512):
    assert M % MB == 0

    def mm_body(x_ref, w_ref, b_ref, o_ref):
        o_ref[...] = (
            lax.dot_general(
                x_ref[...], w_ref[...],
                (((1,), (0,)), ((), ())),
                preferred_element_type=jnp.float32,
            )
            + b_ref[...]
        )

    call = pl.pallas_call(
        mm_body,
        grid=(M // MB,),
        in_specs=[
            pl.BlockSpec((MB, LANES), lambda i: (i, 0)),
            pl.BlockSpec((LANES, V), lambda i: (0, 0)),
            pl.BlockSpec((1, V), lambda i: (0, 0)),
        ],
        out_specs=pl.BlockSpec((MB, V), lambda i: (i, 0)),
        out_shape=jax.ShapeDtypeStruct((M, V), jnp.float32),
    )

    def run(x, W, b):
        wt = jnp.pad(W.T, ((0, LANES - W.shape[1]), (0, 0)))
        return call(x, wt, b.reshape(1, V))

    return run


def kernel(ids, emb_table, W, b):
    B, L = ids.shape
    V, D = emb_table.shape
    ids = ids.astype(jnp.int32)
    grouped = _make_sc_stage(B, L, V, D)(ids, emb_table)
    out = _make_tc_stage(B * L, V, D)(grouped, W, b)
    return out.reshape(B, L, V)


# speedup vs baseline: 1.0459x; 1.0444x over previous
import functools

import jax
import jax.numpy as jnp
from jax import lax
from jax.experimental import pallas as pl
from jax.experimental.pallas import tpu as pltpu
from jax.experimental.pallas import tpu_sc as plsc

WINDOW = 5
LANES = 128


@functools.cache
def _make_sc_stage(B, L, V, D):
    info = plsc.get_sparse_core_info()
    NC, NS = info.num_cores, info.num_subcores
    NW = NC * NS
    NL = info.num_lanes
    assert B % NW == 0 and D == NL
    seq_per_w = B // NW
    rows_per_w = seq_per_w * L
    ids_pad = -(-rows_per_w // LANES) * LANES
    assert rows_per_w % NL == 0
    n_grp = rows_per_w // NL
    SEQ_CHUNK = 8
    assert seq_per_w % SEQ_CHUNK == 0
    n_out_ch = seq_per_w // SEQ_CHUNK
    grp_rows = SEQ_CHUNK * L
    vpad = -(-V * D // LANES)

    mesh = plsc.VectorSubcoreMesh(core_axis_name="c", subcore_axis_name="s")

    @functools.partial(
        pl.kernel,
        mesh=mesh,
        compiler_params=pltpu.CompilerParams(use_tc_tiling_on_sc=True,
                                             needs_layout_passes=False),
        out_type=jax.ShapeDtypeStruct((B * L, LANES), jnp.float32),
        scratch_types=[
            pltpu.VMEM((ids_pad,), jnp.int32),
            pltpu.VMEM((vpad, LANES), jnp.float32),
            pltpu.VMEM((rows_per_w * D,), jnp.float32),
            pltpu.VMEM((L + 1, D), jnp.float32),
            pltpu.VMEM((grp_rows, LANES), jnp.float32),
            pltpu.SemaphoreType.DMA,
        ],
    )
    def sc_kernel(ids_hbm, table_hbm, out_hbm, idx_v, tab_v, rows_v, cum_v,
                  grp_v, sem):
        wid = lax.axis_index("s") * NC + lax.axis_index("c")
        base = wid * rows_per_w

        pltpu.sync_copy(ids_hbm.at[wid], idx_v)
        pltpu.sync_copy(table_hbm, tab_v)

        lane16 = jnp.arange(NL, dtype=jnp.int32) * NL
        zero = jnp.zeros((D,), jnp.float32)

        def zero_body(t, carry):
            for k in range(LANES // D):
                grp_v[t, pl.ds(k * D, D)] = zero
            return carry

        lax.fori_loop(0, grp_rows, zero_body, 0)

        def gather_body(g, carry):
            v = idx_v[pl.ds(g * NL, NL)]
            ri = jax.lax.shift_right_logical(v, 3)
            li0 = jax.lax.shift_left(jnp.bitwise_and(v, 7), 4)
            sbase = lane16 + g * (NL * NL)
            for d in range(D):
                val = plsc.load_gather(tab_v, [ri, li0 + d])
                plsc.store_scatter(rows_v, [sbase + d], val)
            return carry

        lax.fori_loop(0, n_grp, gather_body, 0)

        def seq_body(s8, c):
            row0 = (c * SEQ_CHUNK + s8) * L
            lrow0 = s8 * L
            cum_v[0, :] = zero

            def cum_body(t, acc):
                acc = acc + rows_v[pl.ds((row0 + t) * D, D)]
                cum_v[t + 1, :] = acc
                return acc

            lax.fori_loop(0, L, cum_body, zero)

            def out_body(t, carry):
                hi = jnp.where(t + WINDOW > L, L - 1, t + WINDOW)
                lo = jnp.where(t < WINDOW, 1, t - WINDOW)
                interior = jnp.logical_and(t >= WINDOW, t + WINDOW <= L)
                cmask = jnp.where(interior, 1.0, 0.0).astype(jnp.float32)
                g = (cum_v[hi, :] - cum_v[lo, :]
                     - cmask * rows_v[pl.ds((row0 + t) * D, D)])
                grp_v[lrow0 + t, pl.ds(0, D)] = g
                return carry

            lax.fori_loop(0, L, out_body, 0)
            return c

        for c in range(n_out_ch):
            lax.fori_loop(0, SEQ_CHUNK, seq_body, c)
            pltpu.sync_copy(grp_v, out_hbm.at[pl.ds(base + c * grp_rows,
                                                    grp_rows)])

    def run(ids, table):
        flat = ids.reshape(NW, rows_per_w)
        ids2 = jnp.pad(flat, ((0, 0), (0, ids_pad - rows_per_w)))
        timg = jnp.pad(table.reshape(-1), (0, vpad * LANES - V * D)).reshape(
            vpad, LANES)
        return sc_kernel(ids2, timg)

    return run


@functools.cache
def _make_tc_stage(M, V, D, MB=1024):
    assert M % MB == 0

    def mm_body(x_ref, w_ref, b_ref, o_ref):
        o_ref[...] = (
            lax.dot_general(
                x_ref[...], w_ref[...],
                (((1,), (0,)), ((), ())),
                preferred_element_type=jnp.float32,
            )
            + b_ref[...]
        )

    call = pl.pallas_call(
        mm_body,
        grid=(M // MB,),
        in_specs=[
            pl.BlockSpec((MB, LANES), lambda i: (i, 0)),
            pl.BlockSpec((LANES, V), lambda i: (0, 0)),
            pl.BlockSpec((1, V), lambda i: (0, 0)),
        ],
        out_specs=pl.BlockSpec((MB, V), lambda i: (i, 0)),
        out_shape=jax.ShapeDtypeStruct((M, V), jnp.float32),
    )

    def run(x, W, b):
        wt = jnp.pad(W.T, ((0, LANES - W.shape[1]), (0, 0)))
        return call(x, wt, b.reshape(1, V))

    return run


def kernel(ids, emb_table, W, b):
    B, L = ids.shape
    V, D = emb_table.shape
    ids = ids.astype(jnp.int32)
    grouped = _make_sc_stage(B, L, V, D)(ids, emb_table)
    out = _make_tc_stage(B * L, V, D)(grouped, W, b)
    return out.reshape(B, L, V)
